# baseline (speedup 1.0000x reference)
import functools

import jax
import jax.numpy as jnp
from jax import lax
from jax.experimental import pallas as pl
from jax.experimental.pallas import tpu as pltpu
from jax.experimental.pallas import tpu_sc as plsc

B = 16384
K = 32
NC = 2
NS = 16
L = 16
NW = NC * NS
BPW = B // NW
GROUPS = BPW // L

_mesh = plsc.VectorSubcoreMesh(core_axis_name="c", subcore_axis_name="s")

_f32 = jnp.float32
_i32 = jnp.int32


@functools.partial(
    pl.kernel,
    out_type=(
        jax.ShapeDtypeStruct((B,), _f32),
        jax.ShapeDtypeStruct((B,), _f32),
        jax.ShapeDtypeStruct((B,), _f32),
        jax.ShapeDtypeStruct((B,), _f32),
        jax.ShapeDtypeStruct((B, K), _f32),
        jax.ShapeDtypeStruct((B, K), _f32),
        jax.ShapeDtypeStruct((B, K), _f32),
    ),
    mesh=_mesh,
    compiler_params=pltpu.CompilerParams(
        needs_layout_passes=False, use_tc_tiling_on_sc=False),
    scratch_types=[
        pltpu.VMEM((BPW,), _i32),
        pltpu.VMEM((BPW,), _i32),
        pltpu.VMEM((BPW,), _i32),
        pltpu.VMEM((BPW,), _f32),
        pltpu.VMEM((BPW,), _f32),
        pltpu.VMEM((BPW,), _f32),
        pltpu.VMEM((BPW, K), _f32),
        pltpu.VMEM((BPW, K), _f32),
        pltpu.VMEM((BPW, K), _f32),
        pltpu.VMEM((BPW,), _f32),
        pltpu.SemaphoreType.DMA,
        pltpu.SemaphoreType.DMA,
    ],
)
def _bprmf_sc(user_h, pos_h, neg_h, ubt_h, ibt_h, uet_h, iet_h,
              diff_o, ub_o, ibp_o, ibn_o, ue_o, iep_o, ien_o,
              uidx, pidx, nidx, ubv, ibpv, ibnv, uev, iepv, ienv, diffv,
              gsem, wsem):
    wid = lax.axis_index("s") * NC + lax.axis_index("c")
    base = wid * BPW
    sl = pl.ds(base, BPW)

    pltpu.sync_copy(user_h.at[sl], uidx)
    pltpu.sync_copy(pos_h.at[sl], pidx)
    pltpu.sync_copy(neg_h.at[sl], nidx)

    g = [
        pltpu.async_copy(uet_h.at[uidx], uev, gsem),
        pltpu.async_copy(iet_h.at[pidx], iepv, gsem),
        pltpu.async_copy(iet_h.at[nidx], ienv, gsem),
        pltpu.async_copy(ubt_h.at[uidx], ubv, gsem),
        pltpu.async_copy(ibt_h.at[pidx], ibpv, gsem),
        pltpu.async_copy(ibt_h.at[nidx], ibnv, gsem),
    ]
    for c in g:
        c.wait()

    w = [
        pltpu.async_copy(uev, ue_o.at[sl], wsem),
        pltpu.async_copy(iepv, iep_o.at[sl], wsem),
        pltpu.async_copy(ienv, ien_o.at[sl], wsem),
        pltpu.async_copy(ubv, ub_o.at[sl], wsem),
        pltpu.async_copy(ibpv, ibp_o.at[sl], wsem),
        pltpu.async_copy(ibnv, ibn_o.at[sl], wsem),
    ]

    lane = lax.iota(_i32, L)

    def row_dot(r):
        a0 = uev[r, pl.ds(0, L)]
        a1 = uev[r, pl.ds(L, L)]
        p0 = iepv[r, pl.ds(0, L)]
        p1 = iepv[r, pl.ds(L, L)]
        n0 = ienv[r, pl.ds(0, L)]
        n1 = ienv[r, pl.ds(L, L)]
        t = a0 * (p0 - n0) + a1 * (p1 - n1)
        return jnp.sum(t)

    def group_body(gi, _):
        rbase = gi * L
        acc = ibpv[pl.ds(rbase, L)] - ibnv[pl.ds(rbase, L)]
        for u in range(L):
            s = row_dot(rbase + u)
            acc = acc + jnp.where(lane == u, s, jnp.float32(0.0))
        diffv[pl.ds(rbase, L)] = acc
        return 0

    lax.fori_loop(0, GROUPS, group_body, 0)

    pltpu.sync_copy(diffv, diff_o.at[sl])
    for c in w:
        c.wait()


def kernel(user, pos, neg, offset, user_bias, item_bias, user_emb, item_emb):
    user = user.astype(_i32)
    pos = pos.astype(_i32)
    neg = neg.astype(_i32)
    ubt = user_bias.reshape(-1)
    ibt = item_bias.reshape(-1)

    diff, ub, ibp, ibn, ue, iep, ien = _bprmf_sc(
        user, pos, neg, ubt, ibt, user_emb, item_emb)

    ub2 = ub[:, None]
    pos_params = (offset, ub2, ibp[:, None], ue, iep)
    neg_params = (offset, ub2, ibn[:, None], ue, ien)
    return (diff, pos_params, neg_params)


# speedup vs baseline: 3.0764x; 3.0764x over previous
import functools

import jax
import jax.numpy as jnp
from jax import lax
from jax.experimental import pallas as pl
from jax.experimental.pallas import tpu as pltpu
from jax.experimental.pallas import tpu_sc as plsc

B = 16384
K = 32
NC = 2
NS = 16
L = 16
NW = NC * NS
BPW = B // NW
R = 1000000
WSZ = 1024
NFULL = 976
TAIL0 = 999872
CAP = 1024
RB = 128

_mesh = plsc.VectorSubcoreMesh(core_axis_name="c", subcore_axis_name="s")
_f32 = jnp.float32
_i32 = jnp.int32

_DNUMS = lax.GatherDimensionNumbers(
    offset_dims=(), collapsed_slice_dims=(0,), start_index_map=(0,))


def _dyng(v, u_spl):
    return lax.gather(v, u_spl[:, None], _DNUMS, (1,),
                      mode=lax.GatherScatterMode.PROMISE_IN_BOUNDS)


@functools.partial(
    pl.kernel,
    out_type=(
        jax.ShapeDtypeStruct((B * K,), _f32),
        jax.ShapeDtypeStruct((B * K,), _f32),
        jax.ShapeDtypeStruct((B * K,), _f32),
    ),
    mesh=_mesh,
    compiler_params=pltpu.CompilerParams(
        needs_layout_passes=False, use_tc_tiling_on_sc=True),
    scratch_types=[
        pltpu.VMEM((B,), _i32),
        pltpu.VMEM((CAP,), _i32),
        pltpu.VMEM((CAP,), _i32),
        pltpu.VMEM((CAP,), _i32),
        pltpu.VMEM((CAP,), _i32),
        pltpu.VMEM((CAP,), _i32),
        pltpu.VMEM((CAP,), _i32),
        pltpu.VMEM((K, WSZ), _f32),
        pltpu.VMEM((K, WSZ), _f32),
        pltpu.VMEM((K, 128), _f32),
        pltpu.VMEM((RB * K,), _f32),
        pltpu.SMEM((8,), _i32),
        pltpu.SemaphoreType.DMA,
        pltpu.SemaphoreType.DMA,
        pltpu.SemaphoreType.DMA,
    ],
)
def _bprmf_sweep(user_h, pos_h, neg_h, uet_h, iet_h, tailu_h, taili_h,
                 uef_o, ipf_o, inf_o,
                 ibuf, ur, up_, pr, pp_, nr, np_,
                 winA, winB, tailb, rb, cnts, semA, semB, wsem):
    wid = lax.axis_index("s") * NC + lax.axis_index("c")
    iota = lax.iota(_i32, L)
    cnts[0] = 0
    cnts[1] = 0

    pltpu.async_copy(uet_h.at[:, pl.ds(wid * WSZ, WSZ)], winA, semA)
    pltpu.async_copy(uet_h.at[:, pl.ds((wid + NW) * WSZ, WSZ)], winB, semB)

    def scan(a_h, lr, lp):
        pltpu.sync_copy(a_h, ibuf)

        def ch(c, cnt):
            iv = ibuf[pl.ds(c * L, L)]
            m = ((iv >> 10) & (NW - 1)) == wid
            pc = plsc.all_reduce_population_count(m)[0]

            @pl.when(pc > 0)
            def _():
                csum = plsc.cumsum(jnp.where(m, 1, 0).astype(_i32))
                slots = jnp.where(m, cnt + csum - 1, 0)
                plsc.store_scatter(lr, [slots], iv, mask=m)
                plsc.store_scatter(lp, [slots], c * L + iota, mask=m)

            return cnt + pc

        return lax.fori_loop(0, B // L, ch, jnp.int32(0))

    cu = scan(user_h, ur, up_)
    cp = scan(pos_h, pr, pp_)
    cn = scan(neg_h, nr, np_)

    def extract(buf, lr, lp, cnt, out_h, jsel, lo, hi, coff):
        nch = (cnt + L - 1) >> 4

        def ch(c, _):
            rv = lr[pl.ds(c * L, L)]
            pv = lp[pl.ds(c * L, L)]
            valid = (c * L + iota) < cnt
            m0 = valid & ((rv >> 15) == jsel) & (rv >= lo) & (rv < hi)

            def wcond(cr):
                return plsc.all_reduce_population_count(cr[0] > 0)[0] > 0

            def wbody(cr):
                cm = cr[0]
                u_spl = plsc.all_reduce_ffs(cm > 0)
                r_spl = _dyng(rv, u_spl)
                p_s = _dyng(pv, u_spl)[0]
                col = (r_spl & (WSZ - 1)) - coff
                v0 = plsc.load_gather(buf, [iota, col])
                v1 = plsc.load_gather(buf, [iota + L, col])
                slot = (cnts[0] & (RB - 1)) * K
                rb[pl.ds(slot, L)] = v0
                rb[pl.ds(slot + L, L)] = v1
                pltpu.async_copy(rb.at[pl.ds(slot, K)],
                                 out_h.at[pl.ds(p_s * K, K)], wsem)
                cnts[0] = cnts[0] + 1
                return (cm & (iota != u_spl),)

            lax.while_loop(wcond, wbody, (jnp.where(m0, 1, 0).astype(_i32),))
            return 0

        lax.fori_loop(0, nch, ch, 0)

    def drain(out_h):
        n = cnts[0] - cnts[1]

        def dr(i, _):
            pltpu.make_async_copy(out_h.at[pl.ds(0, K)],
                                  rb.at[pl.ds(0, K)], wsem).wait()
            return 0

        lax.fori_loop(0, n, dr, 0)
        cnts[1] = cnts[0]

    def sweep(tab_h, tail_h, specs, prime):
        if prime:
            pltpu.async_copy(tab_h.at[:, pl.ds(wid * WSZ, WSZ)], winA, semA)
            pltpu.async_copy(
                tab_h.at[:, pl.ds((wid + NW) * WSZ, WSZ)], winB, semB)

        def process(buf, sem, j):
            win_id = wid + NW * j
            pltpu.make_async_copy(tab_h.at[:, pl.ds(0, WSZ)], buf, sem).wait()
            for (lr, lp, cnt, out_h) in specs:
                extract(buf, lr, lp, cnt, out_h, j, 0, R, 0)
            drain(specs[0][3])
            nid = win_id + 2 * NW

            @pl.when(nid <= NFULL - 1)
            def _():
                pltpu.async_copy(tab_h.at[:, pl.ds(nid * WSZ, WSZ)], buf, sem)

        def win_body(j, _):
            win_id = wid + NW * j

            @pl.when(win_id <= NFULL - 1)
            def _():
                @pl.when((j & 1) == 0)
                def _():
                    process(winA, semA, j)

                @pl.when((j & 1) == 1)
                def _():
                    process(winB, semB, j)

            return 0

        lax.fori_loop(0, 31, win_body, 0)

        @pl.when(wid == NFULL % NW)
        def _():
            pltpu.async_copy(tab_h.at[:, pl.ds(NFULL * WSZ, 512)],
                             winA.at[:, pl.ds(0, 512)], semA)
            pltpu.sync_copy(tail_h, tailb)
            pltpu.make_async_copy(tab_h.at[:, pl.ds(0, 512)],
                                  winA.at[:, pl.ds(0, 512)], semA).wait()
            for (lr, lp, cnt, out_h) in specs:
                extract(winA, lr, lp, cnt, out_h, NFULL >> 5,
                        NFULL * WSZ, TAIL0 + 64, 0)
                extract(tailb, lr, lp, cnt, out_h, NFULL >> 5,
                        TAIL0 + 64, R, 448)
            drain(specs[0][3])

    sweep(uet_h, tailu_h, [(ur, up_, cu, uef_o)], prime=False)
    sweep(iet_h, taili_h, [(pr, pp_, cp, ipf_o), (nr, np_, cn, inf_o)],
          prime=True)


@functools.partial(
    pl.kernel,
    out_type=(
        jax.ShapeDtypeStruct((B,), _f32),
        jax.ShapeDtypeStruct((B,), _f32),
        jax.ShapeDtypeStruct((B,), _f32),
        jax.ShapeDtypeStruct((B,), _f32),
    ),
    mesh=_mesh,
    compiler_params=pltpu.CompilerParams(
        needs_layout_passes=False, use_tc_tiling_on_sc=False),
    scratch_types=[
        pltpu.VMEM((BPW,), _i32),
        pltpu.VMEM((BPW,), _i32),
        pltpu.VMEM((BPW,), _i32),
        pltpu.VMEM((BPW,), _f32),
        pltpu.VMEM((BPW,), _f32),
        pltpu.VMEM((BPW,), _f32),
        pltpu.VMEM((BPW * K,), _f32),
        pltpu.VMEM((BPW * K,), _f32),
        pltpu.VMEM((BPW * K,), _f32),
        pltpu.VMEM((BPW,), _f32),
        pltpu.SemaphoreType.DMA,
        pltpu.SemaphoreType.DMA,
    ],
)
def _bprmf_dots(user_h, pos_h, neg_h, ubt_h, ibt_h, uef_h, ipf_h, inf_h,
                diff_o, ub_o, ibp_o, ibn_o,
                uidx, pidx, nidx, ubv, ibpv, ibnv, uev, iepv, ienv, diffv,
                gsem, wsem):
    wid = lax.axis_index("s") * NC + lax.axis_index("c")
    base = wid * BPW
    sl = pl.ds(base, BPW)

    pltpu.sync_copy(user_h.at[sl], uidx)
    pltpu.sync_copy(pos_h.at[sl], pidx)
    pltpu.sync_copy(neg_h.at[sl], nidx)

    g = [
        pltpu.async_copy(ubt_h.at[uidx], ubv, gsem),
        pltpu.async_copy(ibt_h.at[pidx], ibpv, gsem),
        pltpu.async_copy(ibt_h.at[nidx], ibnv, gsem),
        pltpu.async_copy(uef_h.at[pl.ds(base * K, BPW * K)], uev, gsem),
        pltpu.async_copy(ipf_h.at[pl.ds(base * K, BPW * K)], iepv, gsem),
        pltpu.async_copy(inf_h.at[pl.ds(base * K, BPW * K)], ienv, gsem),
    ]
    for c in g:
        c.wait()

    w = [
        pltpu.async_copy(ubv, ub_o.at[sl], wsem),
        pltpu.async_copy(ibpv, ibp_o.at[sl], wsem),
        pltpu.async_copy(ibnv, ibn_o.at[sl], wsem),
    ]

    lane = lax.iota(_i32, L)

    def row_dot(r):
        a0 = uev[pl.ds(r * K, L)]
        a1 = uev[pl.ds(r * K + L, L)]
        p0 = iepv[pl.ds(r * K, L)]
        p1 = iepv[pl.ds(r * K + L, L)]
        n0 = ienv[pl.ds(r * K, L)]
        n1 = ienv[pl.ds(r * K + L, L)]
        return jnp.sum(a0 * (p0 - n0) + a1 * (p1 - n1))

    def group_body(gi, _):
        rbase = gi * L
        acc = ibpv[pl.ds(rbase, L)] - ibnv[pl.ds(rbase, L)]
        for u in range(L):
            s = row_dot(rbase + u)
            acc = acc + jnp.where(lane == u, s, jnp.float32(0.0))
        diffv[pl.ds(rbase, L)] = acc
        return 0

    lax.fori_loop(0, BPW // L, group_body, 0)

    pltpu.sync_copy(diffv, diff_o.at[sl])
    for c in w:
        c.wait()


def kernel(user, pos, neg, offset, user_bias, item_bias, user_emb, item_emb):
    user = user.astype(_i32)
    pos = pos.astype(_i32)
    neg = neg.astype(_i32)
    uet = user_emb.T
    iet = item_emb.T
    tailu = lax.slice(uet, (0, TAIL0), (K, R))
    taili = lax.slice(iet, (0, TAIL0), (K, R))
    ubt = user_bias.reshape(-1)
    ibt = item_bias.reshape(-1)

    uef, ipf, inf = _bprmf_sweep(user, pos, neg, uet, iet, tailu, taili)
    diff, ub, ibp, ibn = _bprmf_dots(user, pos, neg, ubt, ibt, uef, ipf, inf)

    ue = uef.reshape(B, K)
    iep = ipf.reshape(B, K)
    ien = inf.reshape(B, K)
    ub2 = ub[:, None]
    pos_params = (offset, ub2, ibp[:, None], ue, iep)
    neg_params = (offset, ub2, ibn[:, None], ue, ien)
    return (diff, pos_params, neg_params)
